# Initial kernel scaffold; baseline (speedup 1.0000x reference)
import jax
import jax.numpy as jnp
from jax.experimental import pallas as pl


def kernel(x, weight):
    raise NotImplementedError("write your pallas kernel here")


# speedup vs baseline: 1.6159x; 1.6159x over previous
import functools

import jax
import jax.numpy as jnp
from jax.experimental import pallas as pl
from jax.experimental.pallas import tpu as pltpu

DIM = 2048
N_EXPERTS = 64
TOPK = 8
N_GROUPS = 8
GROUP_SIZE = N_EXPERTS // N_GROUPS
TOPK_GROUPS = 4
BB = 256


def _gate_body(x_ref, w_ref, wout_ref, iout_ref):
    x = x_ref[...]
    w = w_ref[...]
    s = jax.lax.dot_general(
        x, w, (((1,), (1,)), ((), ())), preferred_element_type=jnp.float32
    )
    m = jnp.max(s, axis=-1, keepdims=True)
    e = jnp.exp(s - m)
    p = e / jnp.sum(e, axis=-1, keepdims=True)

    g = jnp.concatenate(
        [
            jnp.max(p[:, GROUP_SIZE * i : GROUP_SIZE * (i + 1)], axis=-1, keepdims=True)
            for i in range(N_GROUPS)
        ],
        axis=-1,
    )

    iota_g = jax.lax.broadcasted_iota(jnp.int32, g.shape, 1)
    gmask = jnp.zeros_like(g)
    for _ in range(TOPK_GROUPS):
        mg = jnp.max(g, axis=-1, keepdims=True)
        amg = jnp.min(
            jnp.where(g == mg, iota_g, N_GROUPS), axis=-1, keepdims=True
        )
        sel = iota_g == amg
        gmask = jnp.where(sel, 1.0, gmask)
        g = jnp.where(sel, -1.0, g)

    pm = jnp.concatenate(
        [
            p[:, GROUP_SIZE * i : GROUP_SIZE * (i + 1)] * gmask[:, i : i + 1]
            for i in range(N_GROUPS)
        ],
        axis=-1,
    )

    iota_e = jax.lax.broadcasted_iota(jnp.int32, pm.shape, 1)
    wcols, icols = [], []
    for _ in range(TOPK):
        mv = jnp.max(pm, axis=-1, keepdims=True)
        ai = jnp.min(jnp.where(pm == mv, iota_e, N_EXPERTS), axis=-1, keepdims=True)
        wcols.append(mv)
        icols.append(ai)
        pm = jnp.where(iota_e == ai, -1.0, pm)

    w8 = jnp.maximum(jnp.concatenate(wcols, axis=-1), 1e-7)
    i8 = jnp.concatenate(icols, axis=-1)
    wout_ref[...] = w8
    iout_ref[...] = i8


@functools.partial(jax.jit, static_argnames=())
def kernel(x, weight):
    B = x.shape[0]
    grid = (B // BB,)
    w8, i8 = pl.pallas_call(
        _gate_body,
        grid=grid,
        in_specs=[
            pl.BlockSpec((BB, DIM), lambda i: (i, 0)),
            pl.BlockSpec((N_EXPERTS, DIM), lambda i: (0, 0)),
        ],
        out_specs=[
            pl.BlockSpec((BB, TOPK), lambda i: (i, 0)),
            pl.BlockSpec((BB, TOPK), lambda i: (i, 0)),
        ],
        out_shape=[
            jax.ShapeDtypeStruct((B, TOPK), jnp.float32),
            jax.ShapeDtypeStruct((B, TOPK), jnp.int32),
        ],
        compiler_params=pltpu.CompilerParams(
            dimension_semantics=("arbitrary",),
        ),
    )(x, weight)
    return w8, i8


# speedup vs baseline: 4.3182x; 2.6724x over previous
import functools

import jax
import jax.numpy as jnp
from jax.experimental import pallas as pl
from jax.experimental.pallas import tpu as pltpu

DIM = 2048
N_EXPERTS = 64
TOPK = 8
N_GROUPS = 8
GROUP_SIZE = N_EXPERTS // N_GROUPS
TOPK_GROUPS = 4
BB = 256


def _gate_body(x_ref, w_ref, wout_ref, iout_ref):
    x = x_ref[...]
    w = w_ref[...]
    s = jax.lax.dot_general(
        w, x, (((1,), (1,)), ((), ())), preferred_element_type=jnp.float32
    )
    m = jnp.max(s, axis=0, keepdims=True)
    e = jnp.exp(s - m)
    p = e / jnp.sum(e, axis=0, keepdims=True)

    g = jnp.concatenate(
        [
            jnp.max(p[GROUP_SIZE * i : GROUP_SIZE * (i + 1)], axis=0, keepdims=True)
            for i in range(N_GROUPS)
        ],
        axis=0,
    )

    iota_g = jax.lax.broadcasted_iota(jnp.int32, g.shape, 0)
    gmask = jnp.zeros_like(g)
    for _ in range(TOPK_GROUPS):
        mg = jnp.max(g, axis=0, keepdims=True)
        amg = jnp.min(jnp.where(g == mg, iota_g, N_GROUPS), axis=0, keepdims=True)
        sel = iota_g == amg
        gmask = jnp.where(sel, 1.0, gmask)
        g = jnp.where(sel, -1.0, g)

    pm = jnp.concatenate(
        [
            p[GROUP_SIZE * i : GROUP_SIZE * (i + 1)] * gmask[i : i + 1]
            for i in range(N_GROUPS)
        ],
        axis=0,
    )

    iota_e = jax.lax.broadcasted_iota(jnp.int32, pm.shape, 0)
    wcols, icols = [], []
    for _ in range(TOPK):
        mv = jnp.max(pm, axis=0, keepdims=True)
        ai = jnp.min(jnp.where(pm == mv, iota_e, N_EXPERTS), axis=0, keepdims=True)
        wcols.append(mv)
        icols.append(ai)
        pm = jnp.where(iota_e == ai, -1.0, pm)

    w8 = jnp.maximum(jnp.concatenate(wcols, axis=0), 1e-7)
    i8 = jnp.concatenate(icols, axis=0)
    wout_ref[...] = w8.T
    iout_ref[...] = i8.T


@functools.partial(jax.jit, static_argnames=())
def kernel(x, weight):
    B = x.shape[0]
    grid = (B // BB,)
    w8, i8 = pl.pallas_call(
        _gate_body,
        grid=grid,
        in_specs=[
            pl.BlockSpec((BB, DIM), lambda i: (i, 0)),
            pl.BlockSpec((N_EXPERTS, DIM), lambda i: (0, 0)),
        ],
        out_specs=[
            pl.BlockSpec((BB, TOPK), lambda i: (i, 0)),
            pl.BlockSpec((BB, TOPK), lambda i: (i, 0)),
        ],
        out_shape=[
            jax.ShapeDtypeStruct((B, TOPK), jnp.float32),
            jax.ShapeDtypeStruct((B, TOPK), jnp.int32),
        ],
        compiler_params=pltpu.CompilerParams(
            dimension_semantics=("arbitrary",),
        ),
    )(x, weight)
    return w8, i8


# speedup vs baseline: 5.5628x; 1.2882x over previous
import functools

import jax
import jax.numpy as jnp
from jax.experimental import pallas as pl
from jax.experimental.pallas import tpu as pltpu

DIM = 2048
N_EXPERTS = 64
TOPK = 8
N_GROUPS = 8
GROUP_SIZE = N_EXPERTS // N_GROUPS
TOPK_GROUPS = 4
BB = 512


def _gate_body(x_ref, w_ref, wout_ref, iout_ref):
    x = x_ref[...]
    w = w_ref[...]
    s = jax.lax.dot_general(
        w, x, (((1,), (1,)), ((), ())), preferred_element_type=jnp.float32
    )
    m = jnp.max(s, axis=0, keepdims=True)
    e = jnp.exp(s - m)
    p = e / jnp.sum(e, axis=0, keepdims=True)

    g = jnp.concatenate(
        [
            jnp.max(p[GROUP_SIZE * i : GROUP_SIZE * (i + 1)], axis=0, keepdims=True)
            for i in range(N_GROUPS)
        ],
        axis=0,
    )

    iota_g = jax.lax.broadcasted_iota(jnp.int32, g.shape, 0)
    gmask = jnp.zeros_like(g)
    for _ in range(TOPK_GROUPS):
        mg = jnp.max(g, axis=0, keepdims=True)
        amg = jnp.min(jnp.where(g == mg, iota_g, N_GROUPS), axis=0, keepdims=True)
        sel = iota_g == amg
        gmask = jnp.where(sel, 1.0, gmask)
        g = jnp.where(sel, -1.0, g)

    pm = jnp.concatenate(
        [
            p[GROUP_SIZE * i : GROUP_SIZE * (i + 1)] * gmask[i : i + 1]
            for i in range(N_GROUPS)
        ],
        axis=0,
    )

    iota_e = jax.lax.broadcasted_iota(jnp.int32, pm.shape, 0)
    wcols, icols = [], []
    for _ in range(TOPK):
        mv = jnp.max(pm, axis=0, keepdims=True)
        ai = jnp.min(jnp.where(pm == mv, iota_e, N_EXPERTS), axis=0, keepdims=True)
        wcols.append(mv)
        icols.append(ai)
        pm = jnp.where(iota_e == ai, -1.0, pm)

    w8 = jnp.maximum(jnp.concatenate(wcols, axis=0), 1e-7)
    i8 = jnp.concatenate(icols, axis=0)
    wout_ref[...] = w8.T
    iout_ref[...] = i8.T


@functools.partial(jax.jit, static_argnames=())
def kernel(x, weight):
    B = x.shape[0]
    grid = (B // BB,)
    w8, i8 = pl.pallas_call(
        _gate_body,
        grid=grid,
        in_specs=[
            pl.BlockSpec((BB, DIM), lambda i: (i, 0)),
            pl.BlockSpec((N_EXPERTS, DIM), lambda i: (0, 0)),
        ],
        out_specs=[
            pl.BlockSpec((BB, TOPK), lambda i: (i, 0)),
            pl.BlockSpec((BB, TOPK), lambda i: (i, 0)),
        ],
        out_shape=[
            jax.ShapeDtypeStruct((B, TOPK), jnp.float32),
            jax.ShapeDtypeStruct((B, TOPK), jnp.int32),
        ],
        compiler_params=pltpu.CompilerParams(
            dimension_semantics=("arbitrary",),
        ),
    )(x, weight)
    return w8, i8


# speedup vs baseline: 6.5765x; 1.1822x over previous
import functools

import jax
import jax.numpy as jnp
from jax.experimental import pallas as pl
from jax.experimental.pallas import tpu as pltpu

DIM = 2048
N_EXPERTS = 64
TOPK = 8
N_GROUPS = 8
GROUP_SIZE = N_EXPERTS // N_GROUPS
TOPK_GROUPS = 4
BB = 1024


def _gate_body(x_ref, w_ref, wout_ref, iout_ref):
    x = x_ref[...]
    w = w_ref[...]
    s = jax.lax.dot_general(
        w, x, (((1,), (1,)), ((), ())), preferred_element_type=jnp.float32
    )
    m = jnp.max(s, axis=0, keepdims=True)
    e = jnp.exp(s - m)
    p = e / jnp.sum(e, axis=0, keepdims=True)

    g = jnp.concatenate(
        [
            jnp.max(p[GROUP_SIZE * i : GROUP_SIZE * (i + 1)], axis=0, keepdims=True)
            for i in range(N_GROUPS)
        ],
        axis=0,
    )

    iota_g = jax.lax.broadcasted_iota(jnp.int32, g.shape, 0)
    gmask = jnp.zeros_like(g)
    for _ in range(TOPK_GROUPS):
        mg = jnp.max(g, axis=0, keepdims=True)
        amg = jnp.min(jnp.where(g == mg, iota_g, N_GROUPS), axis=0, keepdims=True)
        sel = iota_g == amg
        gmask = jnp.where(sel, 1.0, gmask)
        g = jnp.where(sel, -1.0, g)

    pm = jnp.concatenate(
        [
            p[GROUP_SIZE * i : GROUP_SIZE * (i + 1)] * gmask[i : i + 1]
            for i in range(N_GROUPS)
        ],
        axis=0,
    )

    iota_e = jax.lax.broadcasted_iota(jnp.int32, pm.shape, 0)
    wcols, icols = [], []
    for _ in range(TOPK):
        mv = jnp.max(pm, axis=0, keepdims=True)
        ai = jnp.min(jnp.where(pm == mv, iota_e, N_EXPERTS), axis=0, keepdims=True)
        wcols.append(mv)
        icols.append(ai)
        pm = jnp.where(iota_e == ai, -1.0, pm)

    w8 = jnp.maximum(jnp.concatenate(wcols, axis=0), 1e-7)
    i8 = jnp.concatenate(icols, axis=0)
    wout_ref[...] = w8.T
    iout_ref[...] = i8.T


@functools.partial(jax.jit, static_argnames=())
def kernel(x, weight):
    B = x.shape[0]
    grid = (B // BB,)
    w8, i8 = pl.pallas_call(
        _gate_body,
        grid=grid,
        in_specs=[
            pl.BlockSpec((BB, DIM), lambda i: (i, 0)),
            pl.BlockSpec((N_EXPERTS, DIM), lambda i: (0, 0)),
        ],
        out_specs=[
            pl.BlockSpec((BB, TOPK), lambda i: (i, 0)),
            pl.BlockSpec((BB, TOPK), lambda i: (i, 0)),
        ],
        out_shape=[
            jax.ShapeDtypeStruct((B, TOPK), jnp.float32),
            jax.ShapeDtypeStruct((B, TOPK), jnp.int32),
        ],
        compiler_params=pltpu.CompilerParams(
            dimension_semantics=("arbitrary",),
        ),
    )(x, weight)
    return w8, i8


# speedup vs baseline: 6.9532x; 1.0573x over previous
import functools

import jax
import jax.numpy as jnp
from jax.experimental import pallas as pl
from jax.experimental.pallas import tpu as pltpu

DIM = 2048
N_EXPERTS = 64
TOPK = 8
N_GROUPS = 8
GROUP_SIZE = N_EXPERTS // N_GROUPS
TOPK_GROUPS = 4
BB = 2048


def _gate_body(x_ref, w_ref, wout_ref, iout_ref):
    x = x_ref[...]
    w = w_ref[...]
    s = jax.lax.dot_general(
        w, x, (((1,), (1,)), ((), ())), preferred_element_type=jnp.float32
    )
    m = jnp.max(s, axis=0, keepdims=True)
    e = jnp.exp(s - m)
    p = e / jnp.sum(e, axis=0, keepdims=True)

    g = jnp.concatenate(
        [
            jnp.max(p[GROUP_SIZE * i : GROUP_SIZE * (i + 1)], axis=0, keepdims=True)
            for i in range(N_GROUPS)
        ],
        axis=0,
    )

    iota_g = jax.lax.broadcasted_iota(jnp.int32, g.shape, 0)
    gmask = jnp.zeros_like(g)
    for _ in range(TOPK_GROUPS):
        mg = jnp.max(g, axis=0, keepdims=True)
        amg = jnp.min(jnp.where(g == mg, iota_g, N_GROUPS), axis=0, keepdims=True)
        sel = iota_g == amg
        gmask = jnp.where(sel, 1.0, gmask)
        g = jnp.where(sel, -1.0, g)

    pm = jnp.concatenate(
        [
            p[GROUP_SIZE * i : GROUP_SIZE * (i + 1)] * gmask[i : i + 1]
            for i in range(N_GROUPS)
        ],
        axis=0,
    )

    iota_e = jax.lax.broadcasted_iota(jnp.int32, pm.shape, 0)
    wcols, icols = [], []
    for _ in range(TOPK):
        mv = jnp.max(pm, axis=0, keepdims=True)
        ai = jnp.min(jnp.where(pm == mv, iota_e, N_EXPERTS), axis=0, keepdims=True)
        wcols.append(mv)
        icols.append(ai)
        pm = jnp.where(iota_e == ai, -1.0, pm)

    w8 = jnp.maximum(jnp.concatenate(wcols, axis=0), 1e-7)
    i8 = jnp.concatenate(icols, axis=0)
    wout_ref[...] = w8.T
    iout_ref[...] = i8.T


@functools.partial(jax.jit, static_argnames=())
def kernel(x, weight):
    B = x.shape[0]
    grid = (B // BB,)
    w8, i8 = pl.pallas_call(
        _gate_body,
        grid=grid,
        in_specs=[
            pl.BlockSpec((BB, DIM), lambda i: (i, 0)),
            pl.BlockSpec((N_EXPERTS, DIM), lambda i: (0, 0)),
        ],
        out_specs=[
            pl.BlockSpec((BB, TOPK), lambda i: (i, 0)),
            pl.BlockSpec((BB, TOPK), lambda i: (i, 0)),
        ],
        out_shape=[
            jax.ShapeDtypeStruct((B, TOPK), jnp.float32),
            jax.ShapeDtypeStruct((B, TOPK), jnp.int32),
        ],
        compiler_params=pltpu.CompilerParams(
            dimension_semantics=("arbitrary",),
        ),
    )(x, weight)
    return w8, i8
